# Initial kernel scaffold; baseline (speedup 1.0000x reference)
import functools

import jax
import jax.numpy as jnp
from jax import lax
from jax.experimental import pallas as pl
from jax.experimental.pallas import tpu as pltpu
from jax.experimental.pallas import tpu_sc as plsc

RES = 128
RADIUS = 1.3
SH_DIM = 9
DATA_DIM = 28
VOXEL = RADIUS * 2 / RES
STEP = VOXEL / 2
N_INTRS = 443
BATCH = 1024
NPTS = BATCH * N_INTRS
C = 32
NVOX = RES * RES * RES

C0 = 0.28209479177387814
C1 = 0.4886025119029199
C2 = (1.0925484305920792, -1.0925484305920792, 0.31539156525252005,
      -1.0925484305920792, 0.5462742152960396)

NSC = 2
NSUB = 16
NW = NSC * NSUB
PPS = NPTS // NW
PB = 32
NIT = PPS // PB


def _ray_bounds(o, d):
    inv = 1.0 / d
    t1 = (-RADIUS - o) * inv
    t2 = (RADIUS - o) * inv
    tnear = jnp.maximum(jnp.max(jnp.minimum(t1, t2), axis=-1), 0.0)
    tfar = jnp.min(jnp.maximum(t1, t2), axis=-1)
    return tnear, tfar


def _prep_body(o_ref, d_ref, idx_ref, w_ref):
    o = o_ref[...]
    d = d_ref[...]
    rb = o.shape[0]
    tnear, tfar = _ray_bounds(o, d)
    k = lax.broadcasted_iota(jnp.float32, (rb, N_INTRS), 1)
    ints0 = tnear[:, None] + k * STEP
    ints1 = tnear[:, None] + (k + 1.0) * STEP
    tmid = 0.5 * (ints0 + ints1)
    mask = tmid < tfar[:, None]
    i0s, i1s, ws = [], [], []
    for a in range(3):
        pa = o[:, a:a + 1] + d[:, a:a + 1] * tmid
        mask = mask & (jnp.abs(pa) <= RADIUS)
        g = (pa / RADIUS + 1.0) * 0.5 * (RES - 1)
        g0 = jnp.floor(g)
        ws.append(g - g0)
        gi = g0.astype(jnp.int32)
        i0s.append(jnp.clip(gi, 0, RES - 1))
        i1s.append(jnp.clip(gi + 1, 0, RES - 1))
    mf = mask.astype(jnp.float32)
    wx, wy, wz = ws
    for j in range(8):
        jx, jy, jz = j & 1, (j >> 1) & 1, (j >> 2) & 1
        ix = i1s[0] if jx else i0s[0]
        iy = i1s[1] if jy else i0s[1]
        iz = i1s[2] if jz else i0s[2]
        wj = ((wx if jx else 1.0 - wx)
              * (wy if jy else 1.0 - wy)
              * (wz if jz else 1.0 - wz) * mf)
        idx_ref[j] = (iz * RES + iy) * RES + ix
        w_ref[j] = wj


def _prep(rays_o, rays_d):
    rb = 128
    grid = (BATCH // rb,)
    return pl.pallas_call(
        _prep_body,
        grid=grid,
        in_specs=[pl.BlockSpec((rb, 3), lambda r: (r, 0)),
                  pl.BlockSpec((rb, 3), lambda r: (r, 0))],
        out_specs=[pl.BlockSpec((8, rb, N_INTRS), lambda r: (0, r, 0)),
                   pl.BlockSpec((8, rb, N_INTRS), lambda r: (0, r, 0))],
        out_shape=[jax.ShapeDtypeStruct((8, BATCH, N_INTRS), jnp.int32),
                   jax.ShapeDtypeStruct((8, BATCH, N_INTRS), jnp.float32)],
    )(rays_o, rays_d)


def _sc_interp(table, idx, w):
    mesh = plsc.VectorSubcoreMesh(core_axis_name="c", subcore_axis_name="s")

    @functools.partial(
        pl.kernel, mesh=mesh,
        out_type=jax.ShapeDtypeStruct((NPTS, C), jnp.float32),
        scratch_types=[
            pltpu.VMEM((8, PB), jnp.int32),
            pltpu.VMEM((8, PB, C), jnp.float32),
            pltpu.VMEM((8, PB), jnp.float32),
            pltpu.VMEM((PB, C), jnp.float32),
            pltpu.SemaphoreType.DMA,
        ],
    )
    def k(table_h, idx_h, w_h, out_h, idx_v, rows_v, w_v, out_v, sem):
        wid = lax.axis_index("s") * NSC + lax.axis_index("c")
        base = wid * PPS

        def body(i, carry):
            p0 = base + i * PB
            pltpu.sync_copy(idx_h.at[:, pl.ds(p0, PB)], idx_v)
            pltpu.sync_copy(w_h.at[:, pl.ds(p0, PB)], w_v)
            cps = [pltpu.async_copy(table_h.at[idx_v.at[j]], rows_v.at[j], sem)
                   for j in range(8)]
            for cp in cps:
                cp.wait()
            for p in range(PB):
                acc0 = w_v[0, p] * rows_v[0, p, 0:16]
                acc1 = w_v[0, p] * rows_v[0, p, 16:32]
                for j in range(1, 8):
                    wj = w_v[j, p]
                    acc0 = acc0 + wj * rows_v[j, p, 0:16]
                    acc1 = acc1 + wj * rows_v[j, p, 16:32]
                out_v[p, 0:16] = acc0
                out_v[p, 16:32] = acc1
            pltpu.sync_copy(out_v, out_h.at[pl.ds(p0, PB)])
            return carry

        lax.fori_loop(0, NIT, body, 0)

    return k(table, idx, w)


def _comp_body(o_ref, d_ref, interp_ref, out_ref):
    o = o_ref[...]
    d = d_ref[...]
    itp = interp_ref[...]
    rb = o.shape[0]
    tnear, _ = _ray_bounds(o, d)
    k = lax.broadcasted_iota(jnp.float32, (rb, N_INTRS), 1)
    ints0 = tnear[:, None] + k * STEP
    ints1 = tnear[:, None] + (k + 1.0) * STEP
    dnorm = jnp.sqrt(jnp.sum(d * d, axis=-1))
    dists = (ints1 - ints0) * dnorm[:, None]

    lane = lax.broadcasted_iota(jnp.int32, (1, 1, C), 2)
    sigma = jnp.sum(jnp.where(lane == DATA_DIM - 1, itp, 0.0), axis=-1)
    sigma = jnp.maximum(sigma, 0.0)
    alpha = 1.0 - jnp.exp(-sigma * dists)
    logt = jnp.log((1.0 - alpha) + 1e-10)
    rr = lax.broadcasted_iota(jnp.int32, (N_INTRS, N_INTRS), 0)
    cc = lax.broadcasted_iota(jnp.int32, (N_INTRS, N_INTRS), 1)
    upper = (rr < cc).astype(jnp.float32)
    cse = jnp.dot(logt, upper, preferred_element_type=jnp.float32)
    abs_light = alpha * jnp.exp(cse)

    x, y, z = d[:, 0], d[:, 1], d[:, 2]
    shm = [jnp.full_like(x, C0), -C1 * y, C1 * z, -C1 * x,
           C2[0] * x * y, C2[1] * y * z,
           C2[2] * (2.0 * z * z - x * x - y * y),
           C2[3] * x * z, C2[4] * (x * x - y * y)]
    shm32 = jnp.zeros((rb, 1, C), jnp.float32)
    for kk in range(SH_DIM):
        for c in range(3):
            shm32 = shm32 + jnp.where(lane == c * SH_DIM + kk,
                                      shm[kk][:, None, None], 0.0)
    prod = itp * shm32
    acc_sum = jnp.sum(abs_light, axis=-1)
    comps = []
    for c in range(3):
        sel = (lane >= c * SH_DIM) & (lane < (c + 1) * SH_DIM)
        rgb = jnp.sum(jnp.where(sel, prod, 0.0), axis=-1)
        comps.append(jnp.sum(abs_light * jax.nn.sigmoid(rgb), axis=-1)
                     + (1.0 - acc_sum))
    out_ref[...] = jnp.stack(comps, axis=-1)


def _comp(rays_o, rays_d, interp):
    rb = 128
    grid = (BATCH // rb,)
    return pl.pallas_call(
        _comp_body,
        grid=grid,
        in_specs=[pl.BlockSpec((rb, 3), lambda r: (r, 0)),
                  pl.BlockSpec((rb, 3), lambda r: (r, 0)),
                  pl.BlockSpec((rb, N_INTRS, C), lambda r: (r, 0, 0))],
        out_specs=pl.BlockSpec((rb, 3), lambda r: (r, 0)),
        out_shape=jax.ShapeDtypeStruct((BATCH, 3), jnp.float32),
    )(rays_o, rays_d, interp)


def kernel(rays_o, rays_d, data):
    d0 = data[0].reshape(DATA_DIM, NVOX)
    table = jnp.pad(d0.T, ((0, 0), (0, C - DATA_DIM)))
    idx, w = _prep(rays_o, rays_d)
    interp = _sc_interp(table, idx.reshape(8, NPTS), w.reshape(8, NPTS))
    return _comp(rays_o, rays_d, interp.reshape(BATCH, N_INTRS, C))


# speedup vs baseline: 1.3710x; 1.3710x over previous
import functools

import jax
import jax.numpy as jnp
from jax import lax
from jax.experimental import pallas as pl
from jax.experimental.pallas import tpu as pltpu
from jax.experimental.pallas import tpu_sc as plsc

RES = 128
RADIUS = 1.3
SH_DIM = 9
DATA_DIM = 28
VOXEL = RADIUS * 2 / RES
STEP = VOXEL / 2
N_INTRS = 443
BATCH = 1024
NPTS = BATCH * N_INTRS
C = 32
NVOX = RES * RES * RES

C0 = 0.28209479177387814
C1 = 0.4886025119029199
C2 = (1.0925484305920792, -1.0925484305920792, 0.31539156525252005,
      -1.0925484305920792, 0.5462742152960396)

NSC = 2
NSUB = 16
NW = NSC * NSUB
PPS = NPTS // NW
PB = 32
NIT = PPS // PB


def _ray_bounds(o, d):
    inv = 1.0 / d
    t1 = (-RADIUS - o) * inv
    t2 = (RADIUS - o) * inv
    tnear = jnp.maximum(jnp.max(jnp.minimum(t1, t2), axis=-1), 0.0)
    tfar = jnp.min(jnp.maximum(t1, t2), axis=-1)
    return tnear, tfar


def _prep_body(o_ref, d_ref, idx_ref, w_ref):
    o = o_ref[...]
    d = d_ref[...]
    rb = o.shape[0]
    tnear, tfar = _ray_bounds(o, d)
    k = lax.broadcasted_iota(jnp.int32, (rb, N_INTRS), 1).astype(jnp.float32)
    ints0 = tnear[:, None] + k * STEP
    ints1 = tnear[:, None] + (k + 1.0) * STEP
    tmid = 0.5 * (ints0 + ints1)
    mask = tmid < tfar[:, None]
    i0s, i1s, ws = [], [], []
    for a in range(3):
        pa = o[:, a:a + 1] + d[:, a:a + 1] * tmid
        mask = mask & (jnp.abs(pa) <= RADIUS)
        g = (pa / RADIUS + 1.0) * 0.5 * (RES - 1)
        g0 = jnp.floor(g)
        ws.append(g - g0)
        gi = g0.astype(jnp.int32)
        i0s.append(jnp.clip(gi, 0, RES - 1))
        i1s.append(jnp.clip(gi + 1, 0, RES - 1))
    mf = mask.astype(jnp.float32)
    wx, wy, wz = ws
    for j in range(8):
        jx, jy, jz = j & 1, (j >> 1) & 1, (j >> 2) & 1
        ix = i1s[0] if jx else i0s[0]
        iy = i1s[1] if jy else i0s[1]
        iz = i1s[2] if jz else i0s[2]
        wj = ((wx if jx else 1.0 - wx)
              * (wy if jy else 1.0 - wy)
              * (wz if jz else 1.0 - wz) * mf)
        idx_ref[j] = (iz * RES + iy) * RES + ix
        w_ref[j] = wj


def _prep(rays_o, rays_d):
    rb = 128
    grid = (BATCH // rb,)
    return pl.pallas_call(
        _prep_body,
        grid=grid,
        in_specs=[pl.BlockSpec((rb, 3), lambda r: (r, 0)),
                  pl.BlockSpec((rb, 3), lambda r: (r, 0))],
        out_specs=[pl.BlockSpec((8, rb, N_INTRS), lambda r: (0, r, 0)),
                   pl.BlockSpec((8, rb, N_INTRS), lambda r: (0, r, 0))],
        out_shape=[jax.ShapeDtypeStruct((8, BATCH, N_INTRS), jnp.int32),
                   jax.ShapeDtypeStruct((8, BATCH, N_INTRS), jnp.float32)],
    )(rays_o, rays_d)


BLK = 8 * PB


def _sc_interp(table, idx, w):
    mesh = plsc.VectorSubcoreMesh(core_axis_name="c", subcore_axis_name="s")

    @functools.partial(
        pl.kernel, mesh=mesh,
        compiler_params=pltpu.CompilerParams(use_tc_tiling_on_sc=False),
        out_type=jax.ShapeDtypeStruct((NPTS, C), jnp.float32),
        scratch_types=[
            pltpu.VMEM((BLK,), jnp.int32),
            pltpu.VMEM((BLK,), jnp.float32),
            pltpu.VMEM((8, PB, C), jnp.float32),
            pltpu.VMEM((PB, C), jnp.float32),
            pltpu.SemaphoreType.DMA,
        ],
    )
    def k(table_h, idx_h, w_h, out_h, idx_v, w_v, rows_v, out_v, sem):
        wid = lax.axis_index("s") * NSC + lax.axis_index("c")
        base = wid * PPS

        def body(i, carry):
            p0 = base + i * PB
            blk = wid * NIT + i
            pltpu.sync_copy(idx_h.at[pl.ds(blk * BLK, BLK)], idx_v)
            pltpu.sync_copy(w_h.at[pl.ds(blk * BLK, BLK)], w_v)
            cps = [pltpu.async_copy(table_h.at[idx_v.at[pl.ds(j * PB, PB)]],
                                    rows_v.at[j], sem)
                   for j in range(8)]
            for cp in cps:
                cp.wait()
            for g in range(PB // 16):
                wvecs = [w_v[pl.ds(j * PB + g * 16, 16)] for j in range(8)]
                for pp in range(16):
                    p = g * 16 + pp
                    acc0 = wvecs[0][pp] * rows_v[0, p, 0:16]
                    acc1 = wvecs[0][pp] * rows_v[0, p, 16:32]
                    for j in range(1, 8):
                        wj = wvecs[j][pp]
                        acc0 = acc0 + wj * rows_v[j, p, 0:16]
                        acc1 = acc1 + wj * rows_v[j, p, 16:32]
                    out_v[p, 0:16] = acc0
                    out_v[p, 16:32] = acc1
            pltpu.sync_copy(out_v, out_h.at[pl.ds(p0, PB)])
            return carry

        lax.fori_loop(0, NIT, body, 0)

    return k(table, idx, w)


def _comp_body(o_ref, d_ref, itp_ref, out_ref):
    o = o_ref[...]
    d = d_ref[...]
    rb = o.shape[0]
    tnear, _ = _ray_bounds(o, d)
    k = lax.broadcasted_iota(jnp.int32, (rb, N_INTRS), 1).astype(jnp.float32)
    ints0 = tnear[:, None] + k * STEP
    ints1 = tnear[:, None] + (k + 1.0) * STEP
    dnorm = jnp.sqrt(jnp.sum(d * d, axis=-1))
    dists = (ints1 - ints0) * dnorm[:, None]

    sigma = jnp.maximum(itp_ref[DATA_DIM - 1], 0.0)
    alpha = 1.0 - jnp.exp(-sigma * dists)
    logt = jnp.log((1.0 - alpha) + 1e-10)
    rr = lax.broadcasted_iota(jnp.int32, (N_INTRS, N_INTRS), 0)
    cc = lax.broadcasted_iota(jnp.int32, (N_INTRS, N_INTRS), 1)
    upper = (rr < cc).astype(jnp.float32)
    cse = jnp.dot(logt, upper, preferred_element_type=jnp.float32)
    abs_light = alpha * jnp.exp(cse)

    x, y, z = d[:, 0], d[:, 1], d[:, 2]
    shm = [jnp.full_like(x, C0), -C1 * y, C1 * z, -C1 * x,
           C2[0] * x * y, C2[1] * y * z,
           C2[2] * (2.0 * z * z - x * x - y * y),
           C2[3] * x * z, C2[4] * (x * x - y * y)]
    acc_sum = jnp.sum(abs_light, axis=-1)
    comps = []
    for c in range(3):
        rgb = shm[0][:, None] * itp_ref[c * SH_DIM]
        for kk in range(1, SH_DIM):
            rgb = rgb + shm[kk][:, None] * itp_ref[c * SH_DIM + kk]
        comps.append(jnp.sum(abs_light * jax.nn.sigmoid(rgb), axis=-1)
                     + (1.0 - acc_sum))
    out_ref[...] = jnp.stack(comps, axis=-1)


def _comp(rays_o, rays_d, interp_t):
    rb = 128
    grid = (BATCH // rb,)
    return pl.pallas_call(
        _comp_body,
        grid=grid,
        in_specs=[pl.BlockSpec((rb, 3), lambda r: (r, 0)),
                  pl.BlockSpec((rb, 3), lambda r: (r, 0)),
                  pl.BlockSpec((DATA_DIM, rb, N_INTRS), lambda r: (0, r, 0))],
        out_specs=pl.BlockSpec((rb, 3), lambda r: (r, 0)),
        out_shape=jax.ShapeDtypeStruct((BATCH, 3), jnp.float32),
    )(rays_o, rays_d, interp_t)


def _iter_major(a):
    return a.reshape(8, NW, NIT, PB).transpose(1, 2, 0, 3).reshape(-1)


def kernel(rays_o, rays_d, data):
    d0 = data[0].reshape(DATA_DIM, NVOX)
    table = jnp.pad(d0.T, ((0, 0), (0, C - DATA_DIM)))
    idx, w = _prep(rays_o, rays_d)
    interp = _sc_interp(table, _iter_major(idx), _iter_major(w))
    interp_t = interp.T[:DATA_DIM].reshape(DATA_DIM, BATCH, N_INTRS)
    return _comp(rays_o, rays_d, interp_t)
